# speedup vs baseline: 2.0128x; 2.0128x over previous
import functools

import jax
import jax.numpy as jnp
from jax import lax
from jax.experimental import pallas as pl
from jax.experimental.pallas import tpu as pltpu
from jax.experimental.pallas import tpu_sc as plsc

_N = 10000
_D = 128
_K = 32
_B = 8192
_NW = 32
_W = 316
_NP = _NW * _W
_C = 4
_RC = _C * _K
_CH = _W // _C
_PPW = _B // _NW
_PC = 64
_PCH = _PPW // _PC

_mesh = plsc.VectorSubcoreMesh(core_axis_name="c", subcore_axis_name="s")


def _wid():
    return lax.axis_index("s") * 2 + lax.axis_index("c")


@functools.partial(
    pl.kernel, mesh=_mesh,
    out_type=jax.ShapeDtypeStruct((_NP, _D), jnp.float32),
    scratch_types=[
        pltpu.VMEM((_CH, _RC), jnp.int32),
        pltpu.VMEM((_RC, _D), jnp.float32),
        pltpu.VMEM((_C, _D), jnp.float32),
        pltpu.SemaphoreType.DMA,
    ],
)
def _layer(nbr_hbm, table_hbm, out_hbm, idx_v, rows_v, acc_v, sem):
    wid = _wid()
    pltpu.sync_copy(nbr_hbm.at[wid], idx_v)

    def chunk(ci, carry):
        pltpu.async_copy(table_hbm.at[idx_v.at[ci]], rows_v, sem).wait()
        for j in range(_C):
            def kstep(k2, accs):
                accs = list(accs)
                for u in range(8):
                    r = j * _K + k2 * 8 + u
                    for g in range(8):
                        accs[g] = accs[g] + rows_v[r, pl.ds(g * 16, 16)]
                return tuple(accs)

            accs = lax.fori_loop(
                0, _K // 8, kstep,
                tuple(jnp.zeros((16,), jnp.float32) for _ in range(8)))
            for g in range(8):
                acc_v[j, pl.ds(g * 16, 16)] = accs[g] * (1.0 / _K)
        pltpu.sync_copy(acc_v, out_hbm.at[pl.ds(wid * _W + ci * _C, _C)])
        return carry

    lax.fori_loop(0, _CH, chunk, 0)


@functools.partial(
    pl.kernel, mesh=_mesh,
    out_type=jax.ShapeDtypeStruct((_B, 16), jnp.float32),
    scratch_types=[
        pltpu.VMEM((_PCH, _PC), jnp.int32),
        pltpu.VMEM((_PCH, _PC), jnp.int32),
        pltpu.VMEM((_PC, _D), jnp.float32),
        pltpu.VMEM((_PC, _D), jnp.float32),
        pltpu.VMEM((_PC, _D), jnp.float32),
        pltpu.VMEM((_PC, _D), jnp.float32),
        pltpu.VMEM((_PC, 16), jnp.float32),
        pltpu.SemaphoreType.DMA,
    ],
)
def _pairs(src_hbm, dst_hbm, h1_hbm, h2_hbm, out_hbm,
           sidx_v, didx_v, rs1_v, rd1_v, rs2_v, rd2_v, out_v, sem):
    wid = _wid()
    pltpu.sync_copy(src_hbm.at[wid], sidx_v)
    pltpu.sync_copy(dst_hbm.at[wid], didx_v)

    def chunk(ci, carry):
        c1 = pltpu.async_copy(h1_hbm.at[sidx_v.at[ci]], rs1_v, sem)
        c2 = pltpu.async_copy(h1_hbm.at[didx_v.at[ci]], rd1_v, sem)
        c3 = pltpu.async_copy(h2_hbm.at[sidx_v.at[ci]], rs2_v, sem)
        c4 = pltpu.async_copy(h2_hbm.at[didx_v.at[ci]], rd2_v, sem)
        c1.wait()
        c2.wait()
        c3.wait()
        c4.wait()

        def pstep(p, carry2):
            acc = jnp.zeros((16,), jnp.float32)
            for g in range(8):
                sl = pl.ds(g * 16, 16)
                v1 = rs1_v[p, sl] - rd1_v[p, sl]
                acc = acc + v1 * v1
                v2 = rs2_v[p, sl] - rd2_v[p, sl]
                acc = acc + v2 * v2
            out_v[p, :] = acc
            return carry2

        lax.fori_loop(0, _PC, pstep, 0)
        pltpu.sync_copy(out_v, out_hbm.at[pl.ds(wid * _PPW + ci * _PC, _PC)])
        return carry

    lax.fori_loop(0, _PCH, chunk, 0)


def _bce_body(d16_ref, lbl_ref, out_ref):
    dsum = jnp.sum(d16_ref[...], axis=1, keepdims=True) * (1.0 / (_D * 2))
    p = jnp.exp(-dsum)
    lbl = lbl_ref[...]
    eps = 1e-7
    t = lbl * jnp.log(p + eps) + (1.0 - lbl) * jnp.log(1.0 - p + eps)
    out_ref[...] = (-jnp.mean(t)).reshape(1, 1)


def kernel(pairs, labels, neighbors, embedding_state):
    nbr3 = jnp.pad(neighbors, ((0, _NP - _N), (0, 0))).reshape(_NW, _CH, _RC)
    h1 = _layer(nbr3, embedding_state)
    h2 = _layer(nbr3, h1)
    src = pairs[:, 0].reshape(_NW, _PCH, _PC)
    dst = pairs[:, 1].reshape(_NW, _PCH, _PC)
    d16 = _pairs(src, dst, h1, h2)
    lblf = labels.astype(jnp.float32).reshape(_B, 1)
    loss = pl.pallas_call(
        _bce_body,
        out_shape=jax.ShapeDtypeStruct((1, 1), jnp.float32),
    )(d16, lblf)
    return loss.reshape(())
